# Initial kernel scaffold; baseline (speedup 1.0000x reference)
import jax
import jax.numpy as jnp
from jax.experimental import pallas as pl


def kernel(hidden, cu_seqlens, term_weight, W1, b1, W2, b2, Ws, bs):
    raise NotImplementedError("write your pallas kernel here")


# speedup vs baseline: 8.1637x; 8.1637x over previous
import functools

import jax
import jax.numpy as jnp
from jax import lax
from jax.experimental import pallas as pl
from jax.experimental.pallas import tpu as pltpu
from jax.experimental.pallas import tpu_sc as plsc

_T = 8192
_H = 256
_S = 8
_D1 = 64
_BT = 512
_NW = 32
_TPW = _T // _NW
_HALO = 8
_LANES = 16


def _tc_stage(h_ref, tw_ref, w1_ref, b1_ref, w2_ref, wsr_ref,
              cu_ref, b2_ref, bs_ref, g_ref, p_ref, se_ref):
    i = pl.program_id(0)
    h = h_ref[...]
    wc = w1_ref[...] * tw_ref[...]
    y = jnp.dot(h, wc, preferred_element_type=jnp.float32) + b1_ref[...]
    s1 = jnp.maximum(y, 0.0)
    g = jnp.sum(s1 * w2_ref[...], axis=1, keepdims=True) + b2_ref[0]
    p = jnp.sum(h * wsr_ref[...], axis=1, keepdims=True) + bs_ref[0]
    pos = i * _BT + lax.broadcasted_iota(jnp.int32, (_BT, 1), 0)
    se = jnp.zeros((_BT, 1), jnp.int32)
    for k in range(1, 9):
        se = jnp.where(pos >= cu_ref[k - 1], cu_ref[k] - 1, se)
    g_ref[...] = g
    p_ref[...] = p
    se_ref[...] = se


def _sc_span(g_hbm, p_hbm, se_hbm, out_hbm, g_v, p_v, se_v, out_v):
    wid = lax.axis_index("s") * 2 + lax.axis_index("c")
    base = wid * _TPW
    pltpu.sync_copy(g_hbm.at[pl.ds(base, _TPW + _HALO)], g_v)
    pltpu.sync_copy(p_hbm.at[pl.ds(base, _TPW + _HALO)], p_v)
    pltpu.sync_copy(se_hbm.at[pl.ds(base, _TPW)], se_v)
    lanes = lax.iota(jnp.int32, _LANES)

    def chunk(c, carry):
        off = c * _LANES
        se = se_v[pl.ds(off, _LANES)]
        pos = base + off + lanes
        m = None
        ssum = None
        ws = None
        for l in range(_S):
            s_l = g_v[pl.ds(off + l, _LANES)]
            v_l = p_v[pl.ds(off + l, _LANES)]
            valid = (pos + l) <= se
            s_l = jnp.where(valid, s_l, jnp.float32(-1e30))
            v_l = jnp.where(valid, v_l, jnp.float32(0.0))
            if l == 0:
                m = s_l
                ssum = jnp.full((_LANES,), 1.0, jnp.float32)
                ws = v_l
            else:
                m2 = jnp.maximum(m, s_l)
                c1 = jnp.exp(m - m2)
                a = jnp.exp(s_l - m2)
                ssum = ssum * c1 + a
                ws = ws * c1 + a * v_l
                m = m2
            out_v[pl.ds(l * _TPW + off, _LANES)] = ws / ssum
        return carry

    lax.fori_loop(0, _TPW // _LANES, chunk, 0)
    pltpu.sync_copy(out_v, out_hbm.at[pl.ds(base * _S, _TPW * _S)])


def kernel(hidden, cu_seqlens, term_weight, W1, b1, W2, b2, Ws, bs):
    grid = _T // _BT
    g, p, se = pl.pallas_call(
        _tc_stage,
        grid=(grid,),
        in_specs=[
            pl.BlockSpec((_BT, _H), lambda i: (i, 0)),
            pl.BlockSpec((_H, 1), lambda i: (0, 0)),
            pl.BlockSpec((_H, _D1), lambda i: (0, 0)),
            pl.BlockSpec((1, _D1), lambda i: (0, 0)),
            pl.BlockSpec((1, _D1), lambda i: (0, 0)),
            pl.BlockSpec((1, _H), lambda i: (0, 0)),
            pl.BlockSpec(memory_space=pltpu.SMEM),
            pl.BlockSpec(memory_space=pltpu.SMEM),
            pl.BlockSpec(memory_space=pltpu.SMEM),
        ],
        out_specs=[
            pl.BlockSpec((_BT, 1), lambda i: (i, 0)),
            pl.BlockSpec((_BT, 1), lambda i: (i, 0)),
            pl.BlockSpec((_BT, 1), lambda i: (i, 0)),
        ],
        out_shape=[
            jax.ShapeDtypeStruct((_T, 1), jnp.float32),
            jax.ShapeDtypeStruct((_T, 1), jnp.float32),
            jax.ShapeDtypeStruct((_T, 1), jnp.int32),
        ],
    )(hidden, term_weight[:, None], W1, b1[None, :], W2.T, Ws.T,
      cu_seqlens, b2, bs)

    g_pad = jnp.pad(g[:, 0], (0, 2 * _HALO))
    p_pad = jnp.pad(p[:, 0], (0, 2 * _HALO))

    sc_call = pl.kernel(
        _sc_span,
        out_type=jax.ShapeDtypeStruct((_T * _S,), jnp.float32),
        mesh=plsc.VectorSubcoreMesh(core_axis_name="c", subcore_axis_name="s"),
        scratch_types=[
            pltpu.VMEM((_TPW + _HALO,), jnp.float32),
            pltpu.VMEM((_TPW + _HALO,), jnp.float32),
            pltpu.VMEM((_TPW,), jnp.int32),
            pltpu.VMEM((_TPW * _S,), jnp.float32),
        ],
    )
    out_flat = sc_call(g_pad, p_pad, se[:, 0])
    return (out_flat.reshape(_NW, _S, _TPW)
            .transpose(0, 2, 1)
            .reshape(_T, _S))
